# Initial kernel scaffold; baseline (speedup 1.0000x reference)
import jax
import jax.numpy as jnp
from jax.experimental import pallas as pl


def kernel(feat, adjs, sampled_nodes, nodes_per_layer, iterations, W1, b1, W2, b2, Wlin, blin):
    raise NotImplementedError("write your pallas kernel here")


# speedup vs baseline: 3.2933x; 3.2933x over previous
import functools

import jax
import jax.numpy as jnp
from jax import lax
from jax.experimental import pallas as pl
from jax.experimental.pallas import tpu as pltpu
from jax.experimental.pallas import tpu_sc as plsc

N = 10000
NP = 10240
D = 128
C = 40
E = 320000
EP = 327680
K = 128
CH = EP // (32 * K)
G = 16
NG = CH // G
NC, NS = 2, 16
RPT = NP // NS
BN = 512


def _make_sc_agg(with_deg):
  mesh = plsc.VectorSubcoreMesh(core_axis_name="c", subcore_axis_name="s")
  out_type = [jax.ShapeDtypeStruct((NC, NP, D), jnp.float32)]
  scratch = [
      pltpu.VMEM((G, K), jnp.int32),
      pltpu.VMEM((G, K), jnp.int32),
      pltpu.VMEM((K, D), jnp.float32),
      pltpu.VMEM((K, D), jnp.float32),
      pltpu.VMEM_SHARED((NP, D), jnp.float32),
      pltpu.SemaphoreType.DMA,
      pltpu.SemaphoreType.DMA,
  ]
  if with_deg:
    out_type.append(jax.ShapeDtypeStruct((NC * NS * NP,), jnp.float32))
    scratch.append(pltpu.VMEM((NP,), jnp.float32))

  def body(y_hbm, src_hbm, dst_hbm, part_hbm, *rest):
    if with_deg:
      degp_hbm, srcv, dstv, rows_a, rows_b, acc, sem_a, sem_b, hist = rest
    else:
      srcv, dstv, rows_a, rows_b, acc, sem_a, sem_b = rest
    cid = lax.axis_index("c")
    sid = lax.axis_index("s")
    gw = cid * NS + sid
    base = sid * RPT

    @pl.loop(0, K)
    def _(i):
      z = jnp.zeros((16,), jnp.float32)
      for j in range(D // 16):
        rows_a[i, pl.ds(j * 16, 16)] = z

    for k in range(RPT // K):
      pltpu.sync_copy(rows_a, acc.at[pl.ds(base + k * K, K)])

    if with_deg:
      @pl.loop(0, NP // 16)
      def _(i):
        hist[pl.ds(i * 16, 16)] = jnp.zeros((16,), jnp.float32)

    plsc.subcore_barrier()

    ones16 = jnp.ones((16,), jnp.float32)

    def count_deg(c):
      for j in range(K // 16):
        idx = dstv[c, pl.ds(j * 16, 16)]
        plsc.addupdate_scatter(hist, [idx], ones16)

    for g in range(NG):
      pltpu.sync_copy(src_hbm.at[pl.ds((gw * NG + g) * G, G)], srcv)
      pltpu.sync_copy(dst_hbm.at[pl.ds((gw * NG + g) * G, G)], dstv)
      pltpu.async_copy(y_hbm.at[srcv.at[0]], rows_a, sem_a)

      @pl.loop(0, G // 2)
      def _(p):
        c0 = 2 * p
        c1 = c0 + 1
        pltpu.make_async_copy(y_hbm.at[srcv.at[c0]], rows_a, sem_a).wait()
        pltpu.async_copy(y_hbm.at[srcv.at[c1]], rows_b, sem_b)
        pltpu.sync_copy(rows_a, acc.at[dstv.at[c0]], add=True)
        if with_deg:
          count_deg(c0)
        pltpu.make_async_copy(y_hbm.at[srcv.at[c1]], rows_b, sem_b).wait()

        @pl.when(c0 + 2 < G)
        def _():
          pltpu.async_copy(y_hbm.at[srcv.at[c0 + 2]], rows_a, sem_a)

        pltpu.sync_copy(rows_b, acc.at[dstv.at[c1]], add=True)
        if with_deg:
          count_deg(c1)

    plsc.subcore_barrier()

    for k in range(RPT // K):
      r = base + k * K
      pltpu.sync_copy(acc.at[pl.ds(r, K)], part_hbm.at[cid, pl.ds(r, K)])
    if with_deg:
      pltpu.sync_copy(hist, degp_hbm.at[pl.ds(gw * NP, NP)])

  return pl.kernel(
      body, out_type=tuple(out_type), mesh=mesh,
      scratch_types=tuple(scratch),
      compiler_params=pltpu.CompilerParams(needs_layout_passes=False))


_sc_agg_deg = _make_sc_agg(True)
_sc_agg = _make_sc_agg(False)


GCH = 8
GK = 40


def _sc_take_body(q_hbm, samp_hbm, out_hbm, sampv, rows_a, rows_b, sem_a,
                  sem_b):
  cid = lax.axis_index("c")
  sid = lax.axis_index("s")
  gw = cid * NS + sid
  pltpu.sync_copy(samp_hbm.at[pl.ds(gw * GCH, GCH)], sampv)
  bufs = [(rows_a, sem_a), (rows_b, sem_b)]
  pltpu.async_copy(q_hbm.at[sampv.at[0]], rows_a, sem_a)
  for c in range(GCH):
    buf, sem = bufs[c % 2]
    pltpu.make_async_copy(q_hbm.at[sampv.at[c]], buf, sem).wait()
    if c + 1 < GCH:
      nbuf, nsem = bufs[(c + 1) % 2]
      pltpu.async_copy(q_hbm.at[sampv.at[c + 1]], nbuf, nsem)
    pltpu.sync_copy(buf, out_hbm.at[pl.ds((gw * GCH + c) * GK, GK)])


_sc_take = pl.kernel(
    _sc_take_body,
    out_type=jax.ShapeDtypeStruct((NP, D), jnp.float32),
    mesh=plsc.VectorSubcoreMesh(core_axis_name="c", subcore_axis_name="s"),
    scratch_types=(
        pltpu.VMEM((GCH, GK), jnp.int32),
        pltpu.VMEM((GK, D), jnp.float32),
        pltpu.VMEM((GK, D), jnp.float32),
        pltpu.SemaphoreType.DMA,
        pltpu.SemaphoreType.DMA,
    ),
)


def _mm_body(x_ref, w_ref, o_ref):
  o_ref[...] = jnp.dot(x_ref[...], w_ref[...],
                       preferred_element_type=jnp.float32)


def _mm(x, w):
  return pl.pallas_call(
      _mm_body,
      grid=(NP // BN,),
      in_specs=[pl.BlockSpec((BN, D), lambda i: (i, 0)),
                pl.BlockSpec((D, D), lambda i: (0, 0))],
      out_specs=pl.BlockSpec((BN, D), lambda i: (i, 0)),
      out_shape=jax.ShapeDtypeStruct((NP, D), jnp.float32),
  )(x, w)


def _agg_to_h(p_ref, dg_ref, b_ref):
  agg = p_ref[0] + p_ref[1]
  deg = jnp.maximum(jnp.sum(dg_ref[...], axis=0), 1.0)[:, None]
  return jnp.maximum(agg / deg + b_ref[...], 0.0)


def _layer_body(p_ref, dg_ref, b_ref, w_ref, o_ref):
  h = _agg_to_h(p_ref, dg_ref, b_ref)
  o_ref[...] = jnp.dot(h, w_ref[...], preferred_element_type=jnp.float32)


def _head_body(p_ref, dg_ref, b_ref, w_ref, bl_ref, o_ref):
  h = _agg_to_h(p_ref, dg_ref, b_ref)
  nrm = jnp.sqrt(jnp.sum(h * h, axis=1, keepdims=True))
  g = h / jnp.maximum(nrm, 1e-12)
  o_ref[...] = jnp.dot(g, w_ref[...],
                       preferred_element_type=jnp.float32) + bl_ref[...]


def _layer(part, degp, b, w):
  return pl.pallas_call(
      _layer_body,
      grid=(NP // BN,),
      in_specs=[pl.BlockSpec((NC, BN, D), lambda i: (0, i, 0)),
                pl.BlockSpec((NC * NS, BN), lambda i: (0, i)),
                pl.BlockSpec((1, D), lambda i: (0, 0)),
                pl.BlockSpec((D, D), lambda i: (0, 0))],
      out_specs=pl.BlockSpec((BN, D), lambda i: (i, 0)),
      out_shape=jax.ShapeDtypeStruct((NP, D), jnp.float32),
  )(part, degp, b, w)


def _head(part, degp, b, w, bl):
  return pl.pallas_call(
      _head_body,
      grid=(NP // BN,),
      in_specs=[pl.BlockSpec((NC, BN, D), lambda i: (0, i, 0)),
                pl.BlockSpec((NC * NS, BN), lambda i: (0, i)),
                pl.BlockSpec((1, D), lambda i: (0, 0)),
                pl.BlockSpec((D, D), lambda i: (0, 0)),
                pl.BlockSpec((1, D), lambda i: (0, 0))],
      out_specs=pl.BlockSpec((BN, D), lambda i: (i, 0)),
      out_shape=jax.ShapeDtypeStruct((NP, D), jnp.float32),
  )(part, degp, b, w, bl)


def kernel(feat, adjs, sampled_nodes, nodes_per_layer, iterations,
           W1, b1, W2, b2, Wlin, blin):
  f32 = jnp.float32
  featp = jnp.zeros((NP, D), f32).at[:N].set(feat)
  src = adjs[0]
  dst = adjs[1]
  srcp = jnp.concatenate(
      [src, jnp.zeros((EP - E,), jnp.int32)]).reshape(EP // K, K)
  dstp = jnp.concatenate(
      [dst, jnp.full((EP - E,), NP - 1, jnp.int32)]).reshape(EP // K, K)
  sampp = jnp.concatenate(
      [sampled_nodes, jnp.zeros((NP - N,), jnp.int32)]).reshape(32 * GCH, GK)
  b1r = b1.reshape(1, D)
  b2r = b2.reshape(1, D)
  wlp = jnp.zeros((D, D), f32).at[:, :C].set(Wlin)
  blp = jnp.zeros((1, D), f32).at[0, :C].set(blin)

  y1 = _mm(featp, W1)
  part1, degp = _sc_agg_deg(y1, srcp, dstp)
  degp = degp.reshape(NC * NS, NP)
  y2 = _layer(part1, degp, b1r, W2)
  part2 = _sc_agg(y2, srcp, dstp)[0]
  q = _head(part2, degp, b2r, wlp, blp)
  outg = _sc_take(q, sampp)
  return outg[:N, :C]


# speedup vs baseline: 3.3925x; 1.0301x over previous
import functools

import jax
import jax.numpy as jnp
from jax import lax
from jax.experimental import pallas as pl
from jax.experimental.pallas import tpu as pltpu
from jax.experimental.pallas import tpu_sc as plsc

N = 10000
NP = 10240
D = 128
C = 40
E = 320000
EP = 327680
K = 128
KC = 64
NCH = EP // (32 * KC)
NC, NS = 2, 16
RPT = NP // NS
BN = 512


def _make_sc_agg(with_deg):
  mesh = plsc.VectorSubcoreMesh(core_axis_name="c", subcore_axis_name="s")
  out_type = [jax.ShapeDtypeStruct((NC, NP, D), jnp.float32)]
  scratch = (
      [pltpu.VMEM((KC, D), jnp.float32) for _ in range(4)]
      + [pltpu.VMEM((KC,), jnp.int32) for _ in range(4)]
      + [pltpu.VMEM((KC,), jnp.int32) for _ in range(4)]
      + [pltpu.VMEM_SHARED((NP, D), jnp.float32)]
      + [pltpu.SemaphoreType.DMA] * 16
  )
  if with_deg:
    out_type.append(jax.ShapeDtypeStruct((NC * NS * NP,), jnp.float32))
    scratch.append(pltpu.VMEM((NP,), jnp.float32))

  def body(y_hbm, src_hbm, dst_hbm, part_hbm, *rest):
    if with_deg:
      degp_hbm = rest[0]
      rest = rest[1:]
      hist = rest[29]
    else:
      hist = None
    rows = rest[0:4]
    srci = rest[4:8]
    dsti = rest[8:12]
    acc = rest[12]
    gsem = rest[13:17]
    ssem = rest[17:21]
    isems = rest[21:25]
    idems = rest[25:29]
    cid = lax.axis_index("c")
    sid = lax.axis_index("s")
    gw = cid * NS + sid
    base = sid * RPT
    ebase = gw * (EP // 32)

    @pl.loop(0, KC)
    def _(i):
      z = jnp.zeros((16,), jnp.float32)
      for j in range(D // 16):
        rows[0][i, pl.ds(j * 16, 16)] = z

    for k in range(RPT // KC):
      pltpu.sync_copy(rows[0], acc.at[pl.ds(base + k * KC, KC)])

    if with_deg:
      @pl.loop(0, NP // 16)
      def _(i):
        hist[pl.ds(i * 16, 16)] = jnp.zeros((16,), jnp.float32)

    plsc.subcore_barrier()

    ones16 = jnp.ones((16,), jnp.float32)

    def count_deg(b):
      for j in range(KC // 16):
        idx = dsti[b][pl.ds(j * 16, 16)]
        plsc.addupdate_scatter(hist, [idx], ones16)

    def load_idx(b, c):
      off = ebase + c * KC
      pltpu.async_copy(src_hbm.at[pl.ds(off, KC)], srci[b], isems[b])
      pltpu.async_copy(dst_hbm.at[pl.ds(off, KC)], dsti[b], idems[b])

    def wait_idx(b):
      pltpu.make_async_copy(src_hbm.at[pl.ds(0, KC)], srci[b],
                            isems[b]).wait()
      pltpu.make_async_copy(dst_hbm.at[pl.ds(0, KC)], dsti[b],
                            idems[b]).wait()

    def issue_gather(b):
      pltpu.async_copy(y_hbm.at[srci[b]], rows[b], gsem[b])

    def wait_gather(b):
      pltpu.make_async_copy(y_hbm.at[srci[b]], rows[b], gsem[b]).wait()

    def issue_scatter(b):
      pltpu.async_copy(rows[b], acc.at[dsti[b]], ssem[b], add=True)

    def wait_scatter(b):
      pltpu.make_async_copy(rows[b], acc.at[dsti[b]], ssem[b]).wait()

    load_idx(0, 0)
    load_idx(1, 1)
    wait_idx(0)
    issue_gather(0)

    @pl.loop(0, NCH // 4)
    def _(p):
      for i in range(4):
        s = 4 * p + i
        b0 = i
        b1 = (i + 1) % 4
        b2 = (i + 2) % 4

        @pl.when(s >= 2)
        def _():
          wait_scatter(b2)

        @pl.when(s + 2 < NCH)
        def _():
          load_idx(b2, s + 2)

        @pl.when(s + 1 < NCH)
        def _():
          wait_idx(b1)
          issue_gather(b1)

        wait_gather(b0)
        issue_scatter(b0)
        if with_deg:
          count_deg(b0)

    wait_scatter((NCH - 2) % 4)
    wait_scatter((NCH - 1) % 4)

    plsc.subcore_barrier()

    for k in range(RPT // K):
      r = base + k * K
      pltpu.sync_copy(acc.at[pl.ds(r, K)], part_hbm.at[cid, pl.ds(r, K)])
    if with_deg:
      pltpu.sync_copy(hist, degp_hbm.at[pl.ds(gw * NP, NP)])

  return pl.kernel(
      body, out_type=tuple(out_type), mesh=mesh,
      scratch_types=tuple(scratch),
      compiler_params=pltpu.CompilerParams(needs_layout_passes=False))


_sc_agg_deg = _make_sc_agg(True)
_sc_agg = _make_sc_agg(False)


GCH = 8
GK = 40


def _sc_take_body(q_hbm, samp_hbm, out_hbm, sampv, rows_a, rows_b, sem_a,
                  sem_b):
  cid = lax.axis_index("c")
  sid = lax.axis_index("s")
  gw = cid * NS + sid
  pltpu.sync_copy(samp_hbm.at[pl.ds(gw * GCH, GCH)], sampv)
  bufs = [(rows_a, sem_a), (rows_b, sem_b)]
  pltpu.async_copy(q_hbm.at[sampv.at[0]], rows_a, sem_a)
  for c in range(GCH):
    buf, sem = bufs[c % 2]
    pltpu.make_async_copy(q_hbm.at[sampv.at[c]], buf, sem).wait()
    if c + 1 < GCH:
      nbuf, nsem = bufs[(c + 1) % 2]
      pltpu.async_copy(q_hbm.at[sampv.at[c + 1]], nbuf, nsem)
    pltpu.sync_copy(buf, out_hbm.at[pl.ds((gw * GCH + c) * GK, GK)])


_sc_take = pl.kernel(
    _sc_take_body,
    out_type=jax.ShapeDtypeStruct((NP, D), jnp.float32),
    mesh=plsc.VectorSubcoreMesh(core_axis_name="c", subcore_axis_name="s"),
    scratch_types=(
        pltpu.VMEM((GCH, GK), jnp.int32),
        pltpu.VMEM((GK, D), jnp.float32),
        pltpu.VMEM((GK, D), jnp.float32),
        pltpu.SemaphoreType.DMA,
        pltpu.SemaphoreType.DMA,
    ),
)


def _mm_body(x_ref, w_ref, o_ref):
  o_ref[...] = jnp.dot(x_ref[...], w_ref[...],
                       preferred_element_type=jnp.float32)


def _mm(x, w):
  return pl.pallas_call(
      _mm_body,
      grid=(NP // BN,),
      in_specs=[pl.BlockSpec((BN, D), lambda i: (i, 0)),
                pl.BlockSpec((D, D), lambda i: (0, 0))],
      out_specs=pl.BlockSpec((BN, D), lambda i: (i, 0)),
      out_shape=jax.ShapeDtypeStruct((NP, D), jnp.float32),
  )(x, w)


def _agg_to_h(p_ref, dg_ref, b_ref):
  agg = p_ref[0] + p_ref[1]
  deg = jnp.maximum(jnp.sum(dg_ref[...], axis=0), 1.0)[:, None]
  return jnp.maximum(agg / deg + b_ref[...], 0.0)


def _layer_body(p_ref, dg_ref, b_ref, w_ref, o_ref):
  h = _agg_to_h(p_ref, dg_ref, b_ref)
  o_ref[...] = jnp.dot(h, w_ref[...], preferred_element_type=jnp.float32)


def _head_body(p_ref, dg_ref, b_ref, w_ref, bl_ref, o_ref):
  h = _agg_to_h(p_ref, dg_ref, b_ref)
  nrm = jnp.sqrt(jnp.sum(h * h, axis=1, keepdims=True))
  g = h / jnp.maximum(nrm, 1e-12)
  o_ref[...] = jnp.dot(g, w_ref[...],
                       preferred_element_type=jnp.float32) + bl_ref[...]


def _layer(part, degp, b, w):
  return pl.pallas_call(
      _layer_body,
      grid=(NP // BN,),
      in_specs=[pl.BlockSpec((NC, BN, D), lambda i: (0, i, 0)),
                pl.BlockSpec((NC * NS, BN), lambda i: (0, i)),
                pl.BlockSpec((1, D), lambda i: (0, 0)),
                pl.BlockSpec((D, D), lambda i: (0, 0))],
      out_specs=pl.BlockSpec((BN, D), lambda i: (i, 0)),
      out_shape=jax.ShapeDtypeStruct((NP, D), jnp.float32),
  )(part, degp, b, w)


def _head(part, degp, b, w, bl):
  return pl.pallas_call(
      _head_body,
      grid=(NP // BN,),
      in_specs=[pl.BlockSpec((NC, BN, D), lambda i: (0, i, 0)),
                pl.BlockSpec((NC * NS, BN), lambda i: (0, i)),
                pl.BlockSpec((1, D), lambda i: (0, 0)),
                pl.BlockSpec((D, D), lambda i: (0, 0)),
                pl.BlockSpec((1, D), lambda i: (0, 0))],
      out_specs=pl.BlockSpec((BN, D), lambda i: (i, 0)),
      out_shape=jax.ShapeDtypeStruct((NP, D), jnp.float32),
  )(part, degp, b, w, bl)


def kernel(feat, adjs, sampled_nodes, nodes_per_layer, iterations,
           W1, b1, W2, b2, Wlin, blin):
  f32 = jnp.float32
  featp = jnp.zeros((NP, D), f32).at[:N].set(feat)
  src = adjs[0]
  dst = adjs[1]
  srcp = jnp.concatenate([src, jnp.zeros((EP - E,), jnp.int32)])
  dstp = jnp.concatenate([dst, jnp.full((EP - E,), NP - 1, jnp.int32)])
  sampp = jnp.concatenate(
      [sampled_nodes, jnp.zeros((NP - N,), jnp.int32)]).reshape(32 * GCH, GK)
  b1r = b1.reshape(1, D)
  b2r = b2.reshape(1, D)
  wlp = jnp.zeros((D, D), f32).at[:, :C].set(Wlin)
  blp = jnp.zeros((1, D), f32).at[0, :C].set(blin)

  y1 = _mm(featp, W1)
  part1, degp = _sc_agg_deg(y1, srcp, dstp)
  degp = degp.reshape(NC * NS, NP)
  y2 = _layer(part1, degp, b1r, W2)
  part2 = _sc_agg(y2, srcp, dstp)[0]
  q = _head(part2, degp, b2r, wlp, blp)
  outg = _sc_take(q, sampp)
  return outg[:N, :C]


# speedup vs baseline: 8.8304x; 2.6029x over previous
import jax
import jax.numpy as jnp
from jax import lax
from jax.experimental import pallas as pl
from jax.experimental.pallas import tpu as pltpu
from jax.experimental.pallas import tpu_sc as plsc

N = 10000
NP = 10240
D = 128
HW = D // 2
C = 40
E = 320000
EP = 327680
KC = 128
NC, NS = 2, 16
NCH = EP // (NS * KC)
RPT = NP // NS
BN = 512


def _make_sc_agg(with_deg):
  mesh = plsc.VectorSubcoreMesh(core_axis_name="c", subcore_axis_name="s")
  out_type = [jax.ShapeDtypeStruct((NC, NP, HW), jnp.float32)]
  scratch = (
      [pltpu.VMEM((KC, HW), jnp.float32) for _ in range(4)]
      + [pltpu.VMEM((KC,), jnp.int32) for _ in range(4)]
      + [pltpu.VMEM((KC,), jnp.int32) for _ in range(4)]
      + [pltpu.VMEM_SHARED((NP, HW), jnp.float32)]
      + [pltpu.VMEM_SHARED((NP, HW), jnp.float32)]
      + [pltpu.SemaphoreType.DMA] * 16
  )
  if with_deg:
    out_type.append(jax.ShapeDtypeStruct((NC * NS * NP,), jnp.float32))
    scratch.append(pltpu.VMEM((NP,), jnp.float32))

  def body(y_hbm, src_hbm, dst_hbm, part_hbm, *rest):
    if with_deg:
      degp_hbm = rest[0]
      rest = rest[1:]
      hist = rest[30]
    else:
      hist = None
    rows = rest[0:4]
    srci = rest[4:8]
    dsti = rest[8:12]
    acc = rest[12]
    y_sp = rest[13]
    gsem = rest[14:18]
    ssem = rest[18:22]
    isems = rest[22:26]
    idems = rest[26:30]
    cid = lax.axis_index("c")
    sid = lax.axis_index("s")
    gw = cid * NS + sid
    base = sid * RPT
    ebase = sid * (EP // NS)

    pltpu.sync_copy(y_hbm.at[cid, pl.ds(base, RPT)],
                    y_sp.at[pl.ds(base, RPT)])

    @pl.loop(0, KC)
    def _(i):
      z = jnp.zeros((16,), jnp.float32)
      for j in range(HW // 16):
        rows[0][i, pl.ds(j * 16, 16)] = z

    for k in range(RPT // KC):
      pltpu.sync_copy(rows[0], acc.at[pl.ds(base + k * KC, KC)])

    if with_deg:
      @pl.loop(0, NP // 16)
      def _(i):
        hist[pl.ds(i * 16, 16)] = jnp.zeros((16,), jnp.float32)

    plsc.subcore_barrier()

    ones16 = jnp.ones((16,), jnp.float32)

    def count_deg(b):
      for j in range(KC // 16):
        idx = dsti[b][pl.ds(j * 16, 16)]
        plsc.addupdate_scatter(hist, [idx], ones16)

    def load_idx(b, c):
      off = ebase + c * KC
      pltpu.async_copy(src_hbm.at[pl.ds(off, KC)], srci[b], isems[b])
      pltpu.async_copy(dst_hbm.at[pl.ds(off, KC)], dsti[b], idems[b])

    def wait_idx(b):
      pltpu.make_async_copy(src_hbm.at[pl.ds(0, KC)], srci[b],
                            isems[b]).wait()
      pltpu.make_async_copy(dst_hbm.at[pl.ds(0, KC)], dsti[b],
                            idems[b]).wait()

    def issue_gather(b):
      pltpu.async_copy(y_sp.at[srci[b]], rows[b], gsem[b])

    def wait_gather(b):
      pltpu.make_async_copy(y_sp.at[srci[b]], rows[b], gsem[b]).wait()

    def issue_scatter(b):
      pltpu.async_copy(rows[b], acc.at[dsti[b]], ssem[b], add=True)

    def wait_scatter(b):
      pltpu.make_async_copy(rows[b], acc.at[dsti[b]], ssem[b]).wait()

    load_idx(0, 0)
    load_idx(1, 1)
    wait_idx(0)
    issue_gather(0)

    @pl.loop(0, NCH // 4)
    def _(p):
      for i in range(4):
        s = 4 * p + i
        b0 = i
        b1 = (i + 1) % 4
        b2 = (i + 2) % 4

        @pl.when(s >= 2)
        def _():
          wait_scatter(b2)

        @pl.when(s + 2 < NCH)
        def _():
          load_idx(b2, s + 2)

        @pl.when(s + 1 < NCH)
        def _():
          wait_idx(b1)
          issue_gather(b1)

        wait_gather(b0)
        issue_scatter(b0)
        if with_deg:
          count_deg(b0)

    wait_scatter((NCH - 2) % 4)
    wait_scatter((NCH - 1) % 4)

    plsc.subcore_barrier()

    for k in range(RPT // KC):
      r = base + k * KC
      pltpu.sync_copy(acc.at[pl.ds(r, KC)], part_hbm.at[cid, pl.ds(r, KC)])
    if with_deg:
      pltpu.sync_copy(hist, degp_hbm.at[pl.ds(gw * NP, NP)])

  return pl.kernel(
      body, out_type=tuple(out_type), mesh=mesh,
      scratch_types=tuple(scratch),
      compiler_params=pltpu.CompilerParams(
          needs_layout_passes=False, use_tc_tiling_on_sc=False))


_sc_agg_deg = _make_sc_agg(True)
_sc_agg = _make_sc_agg(False)


GCH = 8
GK = 40


def _sc_take_body(q_hbm, samp_hbm, out_hbm, sampv, rows_a, rows_b, sem_a,
                  sem_b):
  cid = lax.axis_index("c")
  sid = lax.axis_index("s")
  gw = cid * NS + sid
  pltpu.sync_copy(samp_hbm.at[pl.ds(gw * GCH, GCH)], sampv)
  bufs = [(rows_a, sem_a), (rows_b, sem_b)]
  pltpu.async_copy(q_hbm.at[sampv.at[0]], rows_a, sem_a)
  for c in range(GCH):
    buf, sem = bufs[c % 2]
    pltpu.make_async_copy(q_hbm.at[sampv.at[c]], buf, sem).wait()
    if c + 1 < GCH:
      nbuf, nsem = bufs[(c + 1) % 2]
      pltpu.async_copy(q_hbm.at[sampv.at[c + 1]], nbuf, nsem)
    pltpu.sync_copy(buf, out_hbm.at[pl.ds((gw * GCH + c) * GK, GK)])


_sc_take = pl.kernel(
    _sc_take_body,
    out_type=jax.ShapeDtypeStruct((NP, D), jnp.float32),
    mesh=plsc.VectorSubcoreMesh(core_axis_name="c", subcore_axis_name="s"),
    scratch_types=(
        pltpu.VMEM((GCH, GK), jnp.int32),
        pltpu.VMEM((GK, D), jnp.float32),
        pltpu.VMEM((GK, D), jnp.float32),
        pltpu.SemaphoreType.DMA,
        pltpu.SemaphoreType.DMA,
    ),
)


def _split_cols(y):
  return jnp.stack([y[:, :HW], y[:, HW:]])


def _mm_body(x_ref, w_ref, o_ref):
  y = jnp.dot(x_ref[...], w_ref[...], preferred_element_type=jnp.float32)
  o_ref[...] = _split_cols(y)


def _mm(x, w):
  return pl.pallas_call(
      _mm_body,
      grid=(NP // BN,),
      in_specs=[pl.BlockSpec((BN, D), lambda i: (i, 0)),
                pl.BlockSpec((D, D), lambda i: (0, 0))],
      out_specs=pl.BlockSpec((NC, BN, HW), lambda i: (0, i, 0)),
      out_shape=jax.ShapeDtypeStruct((NC, NP, HW), jnp.float32),
  )(x, w)


def _agg_to_h(p_ref, dg_ref, b_ref):
  agg = jnp.concatenate([p_ref[0], p_ref[1]], axis=1)
  deg = jnp.maximum(0.5 * jnp.sum(dg_ref[...], axis=0), 1.0)[:, None]
  return jnp.maximum(agg / deg + b_ref[...], 0.0)


def _layer_body(p_ref, dg_ref, b_ref, w_ref, o_ref):
  h = _agg_to_h(p_ref, dg_ref, b_ref)
  y = jnp.dot(h, w_ref[...], preferred_element_type=jnp.float32)
  o_ref[...] = _split_cols(y)


def _head_body(p_ref, dg_ref, b_ref, w_ref, bl_ref, o_ref):
  h = _agg_to_h(p_ref, dg_ref, b_ref)
  nrm = jnp.sqrt(jnp.sum(h * h, axis=1, keepdims=True))
  g = h / jnp.maximum(nrm, 1e-12)
  o_ref[...] = jnp.dot(g, w_ref[...],
                       preferred_element_type=jnp.float32) + bl_ref[...]


def _layer(part, degp, b, w):
  return pl.pallas_call(
      _layer_body,
      grid=(NP // BN,),
      in_specs=[pl.BlockSpec((NC, BN, HW), lambda i: (0, i, 0)),
                pl.BlockSpec((NC * NS, BN), lambda i: (0, i)),
                pl.BlockSpec((1, D), lambda i: (0, 0)),
                pl.BlockSpec((D, D), lambda i: (0, 0))],
      out_specs=pl.BlockSpec((NC, BN, HW), lambda i: (0, i, 0)),
      out_shape=jax.ShapeDtypeStruct((NC, NP, HW), jnp.float32),
  )(part, degp, b, w)


def _head(part, degp, b, w, bl):
  return pl.pallas_call(
      _head_body,
      grid=(NP // BN,),
      in_specs=[pl.BlockSpec((NC, BN, HW), lambda i: (0, i, 0)),
                pl.BlockSpec((NC * NS, BN), lambda i: (0, i)),
                pl.BlockSpec((1, D), lambda i: (0, 0)),
                pl.BlockSpec((D, D), lambda i: (0, 0)),
                pl.BlockSpec((1, D), lambda i: (0, 0))],
      out_specs=pl.BlockSpec((BN, D), lambda i: (i, 0)),
      out_shape=jax.ShapeDtypeStruct((NP, D), jnp.float32),
  )(part, degp, b, w, bl)


def kernel(feat, adjs, sampled_nodes, nodes_per_layer, iterations,
           W1, b1, W2, b2, Wlin, blin):
  f32 = jnp.float32
  featp = jnp.zeros((NP, D), f32).at[:N].set(feat)
  src = adjs[0]
  dst = adjs[1]
  srcp = jnp.concatenate([src, jnp.zeros((EP - E,), jnp.int32)])
  dstp = jnp.concatenate([dst, jnp.full((EP - E,), NP - 1, jnp.int32)])
  sampp = jnp.concatenate(
      [sampled_nodes, jnp.zeros((NP - N,), jnp.int32)]).reshape(32 * GCH, GK)
  b1r = b1.reshape(1, D)
  b2r = b2.reshape(1, D)
  wlp = jnp.zeros((D, D), f32).at[:, :C].set(Wlin)
  blp = jnp.zeros((1, D), f32).at[0, :C].set(blin)

  y1 = _mm(featp, W1)
  part1, degp = _sc_agg_deg(y1, srcp, dstp)
  degp = degp.reshape(NC * NS, NP)
  y2 = _layer(part1, degp, b1r, W2)
  part2 = _sc_agg(y2, srcp, dstp)[0]
  q = _head(part2, degp, b2r, wlp, blp)
  outg = _sc_take(q, sampp)
  return outg[:N, :C]


# speedup vs baseline: 9.0970x; 1.0302x over previous
import jax
import jax.numpy as jnp
from jax import lax
from jax.experimental import pallas as pl
from jax.experimental.pallas import tpu as pltpu
from jax.experimental.pallas import tpu_sc as plsc

N = 10000
NP = 10240
D = 128
HW = D // 2
C = 40
E = 320000
EP = 327680
KC = 128
NC, NS = 2, 16
NCH = EP // (NS * KC)
RPT = NP // NS
BN = 512


def _make_sc_agg(with_deg):
  mesh = plsc.VectorSubcoreMesh(core_axis_name="c", subcore_axis_name="s")
  out_type = [jax.ShapeDtypeStruct((NC, NP, HW), jnp.float32)]
  scratch = (
      [pltpu.VMEM((KC, HW), jnp.float32) for _ in range(4)]
      + [pltpu.VMEM((KC,), jnp.int32) for _ in range(4)]
      + [pltpu.VMEM((KC,), jnp.int32) for _ in range(4)]
      + [pltpu.VMEM_SHARED((NP, HW), jnp.float32)]
      + [pltpu.VMEM_SHARED((NP, HW), jnp.float32)]
      + [pltpu.SemaphoreType.DMA] * 16
  )
  if with_deg:
    out_type.append(jax.ShapeDtypeStruct((NC * NS * NP,), jnp.float32))
    scratch.append(pltpu.VMEM((NP,), jnp.float32))

  def body(y_hbm, src_hbm, dst_hbm, part_hbm, *rest):
    if with_deg:
      degp_hbm = rest[0]
      rest = rest[1:]
      hist = rest[30]
    else:
      hist = None
    rows = rest[0:4]
    srci = rest[4:8]
    dsti = rest[8:12]
    acc = rest[12]
    y_sp = rest[13]
    gsem = rest[14:18]
    ssem = rest[18:22]
    isems = rest[22:26]
    idems = rest[26:30]
    cid = lax.axis_index("c")
    sid = lax.axis_index("s")
    gw = cid * NS + sid
    base = sid * RPT
    ebase = sid * (EP // NS)

    pltpu.sync_copy(y_hbm.at[cid, pl.ds(base, RPT)],
                    y_sp.at[pl.ds(base, RPT)])

    @pl.loop(0, KC)
    def _(i):
      z = jnp.zeros((16,), jnp.float32)
      for j in range(HW // 16):
        rows[0][i, pl.ds(j * 16, 16)] = z

    for k in range(RPT // KC):
      pltpu.sync_copy(rows[0], acc.at[pl.ds(base + k * KC, KC)])

    if with_deg:
      @pl.loop(0, NP // 16)
      def _(i):
        hist[pl.ds(i * 16, 16)] = jnp.zeros((16,), jnp.float32)

    plsc.subcore_barrier()

    ones16 = jnp.ones((16,), jnp.float32)

    def count_deg(b):
      for j in range(KC // 16):
        idx = dsti[b][pl.ds(j * 16, 16)]
        plsc.addupdate_scatter(hist, [idx], ones16)

    def load_idx(b, c):
      off = ebase + c * KC
      pltpu.async_copy(src_hbm.at[pl.ds(off, KC)], srci[b], isems[b])
      pltpu.async_copy(dst_hbm.at[pl.ds(off, KC)], dsti[b], idems[b])

    def wait_idx(b):
      pltpu.make_async_copy(src_hbm.at[pl.ds(0, KC)], srci[b],
                            isems[b]).wait()
      pltpu.make_async_copy(dst_hbm.at[pl.ds(0, KC)], dsti[b],
                            idems[b]).wait()

    def issue_gather(b):
      pltpu.async_copy(y_sp.at[srci[b]], rows[b], gsem[b])

    def wait_gather(b):
      pltpu.make_async_copy(y_sp.at[srci[b]], rows[b], gsem[b]).wait()

    def issue_scatter(b):
      pltpu.async_copy(rows[b], acc.at[dsti[b]], ssem[b], add=True)

    def wait_scatter(b):
      pltpu.make_async_copy(rows[b], acc.at[dsti[b]], ssem[b]).wait()

    load_idx(0, 0)
    load_idx(1, 1)
    wait_idx(0)
    issue_gather(0)

    @pl.loop(0, NCH // 4)
    def _(p):
      for i in range(4):
        s = 4 * p + i
        b0 = i
        b1 = (i + 1) % 4
        b2 = (i + 2) % 4

        @pl.when(s >= 2)
        def _():
          wait_scatter(b2)

        @pl.when(s + 2 < NCH)
        def _():
          load_idx(b2, s + 2)

        @pl.when(s + 1 < NCH)
        def _():
          wait_idx(b1)
          issue_gather(b1)

        wait_gather(b0)
        issue_scatter(b0)
        if with_deg:
          count_deg(b0)

    wait_scatter((NCH - 2) % 4)
    wait_scatter((NCH - 1) % 4)

    plsc.subcore_barrier()

    for k in range(RPT // KC):
      r = base + k * KC
      pltpu.sync_copy(acc.at[pl.ds(r, KC)], part_hbm.at[cid, pl.ds(r, KC)])
    if with_deg:
      pltpu.sync_copy(hist, degp_hbm.at[pl.ds(gw * NP, NP)])

  return pl.kernel(
      body, out_type=tuple(out_type), mesh=mesh,
      scratch_types=tuple(scratch),
      compiler_params=pltpu.CompilerParams(
          needs_layout_passes=False, use_tc_tiling_on_sc=False))


_sc_agg_deg = _make_sc_agg(True)
_sc_agg = _make_sc_agg(False)


GCH = 8
GK = 40


def _sc_take_body(q_hbm, samp_hbm, out_hbm, sampv, rows_a, rows_b, q_sp,
                  sem_a, sem_b):
  cid = lax.axis_index("c")
  sid = lax.axis_index("s")
  gw = cid * NS + sid
  base = sid * RPT
  pltpu.sync_copy(q_hbm.at[pl.ds(base, RPT)], q_sp.at[pl.ds(base, RPT)])
  pltpu.sync_copy(samp_hbm.at[pl.ds(gw * GCH, GCH)], sampv)
  plsc.subcore_barrier()
  bufs = [(rows_a, sem_a), (rows_b, sem_b)]
  pltpu.async_copy(q_sp.at[sampv.at[0]], rows_a, sem_a)
  for c in range(GCH):
    buf, sem = bufs[c % 2]
    pltpu.make_async_copy(q_sp.at[sampv.at[c]], buf, sem).wait()
    if c + 1 < GCH:
      nbuf, nsem = bufs[(c + 1) % 2]
      pltpu.async_copy(q_sp.at[sampv.at[c + 1]], nbuf, nsem)
    pltpu.sync_copy(buf, out_hbm.at[pl.ds((gw * GCH + c) * GK, GK)])


_sc_take = pl.kernel(
    _sc_take_body,
    out_type=jax.ShapeDtypeStruct((NP, D), jnp.float32),
    mesh=plsc.VectorSubcoreMesh(core_axis_name="c", subcore_axis_name="s"),
    scratch_types=(
        pltpu.VMEM((GCH, GK), jnp.int32),
        pltpu.VMEM((GK, D), jnp.float32),
        pltpu.VMEM((GK, D), jnp.float32),
        pltpu.VMEM_SHARED((NP, D), jnp.float32),
        pltpu.SemaphoreType.DMA,
        pltpu.SemaphoreType.DMA,
    ),
    compiler_params=pltpu.CompilerParams(
        needs_layout_passes=False, use_tc_tiling_on_sc=False))


def _split_cols(y):
  return jnp.stack([y[:, :HW], y[:, HW:]])


def _mm_body(x_ref, w_ref, o_ref):
  y = jnp.dot(x_ref[...], w_ref[...], preferred_element_type=jnp.float32)
  o_ref[...] = _split_cols(y)


def _mm(x, w):
  return pl.pallas_call(
      _mm_body,
      grid=(NP // BN,),
      in_specs=[pl.BlockSpec((BN, D), lambda i: (i, 0)),
                pl.BlockSpec((D, D), lambda i: (0, 0))],
      out_specs=pl.BlockSpec((NC, BN, HW), lambda i: (0, i, 0)),
      out_shape=jax.ShapeDtypeStruct((NC, NP, HW), jnp.float32),
  )(x, w)


def _agg_to_h(p_ref, dg_ref, b_ref):
  agg = jnp.concatenate([p_ref[0], p_ref[1]], axis=1)
  deg = jnp.maximum(0.5 * jnp.sum(dg_ref[...], axis=0), 1.0)[:, None]
  return jnp.maximum(agg / deg + b_ref[...], 0.0)


def _layer_body(p_ref, dg_ref, b_ref, w_ref, o_ref):
  h = _agg_to_h(p_ref, dg_ref, b_ref)
  y = jnp.dot(h, w_ref[...], preferred_element_type=jnp.float32)
  o_ref[...] = _split_cols(y)


def _head_body(p_ref, dg_ref, b_ref, w_ref, bl_ref, o_ref):
  h = _agg_to_h(p_ref, dg_ref, b_ref)
  nrm = jnp.sqrt(jnp.sum(h * h, axis=1, keepdims=True))
  g = h / jnp.maximum(nrm, 1e-12)
  o_ref[...] = jnp.dot(g, w_ref[...],
                       preferred_element_type=jnp.float32) + bl_ref[...]


def _layer(part, degp, b, w):
  return pl.pallas_call(
      _layer_body,
      grid=(NP // BN,),
      in_specs=[pl.BlockSpec((NC, BN, HW), lambda i: (0, i, 0)),
                pl.BlockSpec((NC * NS, BN), lambda i: (0, i)),
                pl.BlockSpec((1, D), lambda i: (0, 0)),
                pl.BlockSpec((D, D), lambda i: (0, 0))],
      out_specs=pl.BlockSpec((NC, BN, HW), lambda i: (0, i, 0)),
      out_shape=jax.ShapeDtypeStruct((NC, NP, HW), jnp.float32),
  )(part, degp, b, w)


def _head(part, degp, b, w, bl):
  return pl.pallas_call(
      _head_body,
      grid=(NP // BN,),
      in_specs=[pl.BlockSpec((NC, BN, HW), lambda i: (0, i, 0)),
                pl.BlockSpec((NC * NS, BN), lambda i: (0, i)),
                pl.BlockSpec((1, D), lambda i: (0, 0)),
                pl.BlockSpec((D, D), lambda i: (0, 0)),
                pl.BlockSpec((1, D), lambda i: (0, 0))],
      out_specs=pl.BlockSpec((BN, D), lambda i: (i, 0)),
      out_shape=jax.ShapeDtypeStruct((NP, D), jnp.float32),
  )(part, degp, b, w, bl)


def kernel(feat, adjs, sampled_nodes, nodes_per_layer, iterations,
           W1, b1, W2, b2, Wlin, blin):
  f32 = jnp.float32
  featp = jnp.zeros((NP, D), f32).at[:N].set(feat)
  src = adjs[0]
  dst = adjs[1]
  srcp = jnp.concatenate([src, jnp.zeros((EP - E,), jnp.int32)])
  dstp = jnp.concatenate([dst, jnp.full((EP - E,), NP - 1, jnp.int32)])
  sampp = jnp.concatenate(
      [sampled_nodes, jnp.zeros((NP - N,), jnp.int32)]).reshape(32 * GCH, GK)
  b1r = b1.reshape(1, D)
  b2r = b2.reshape(1, D)
  wlp = jnp.zeros((D, D), f32).at[:, :C].set(Wlin)
  blp = jnp.zeros((1, D), f32).at[0, :C].set(blin)

  y1 = _mm(featp, W1)
  part1, degp = _sc_agg_deg(y1, srcp, dstp)
  degp = degp.reshape(NC * NS, NP)
  y2 = _layer(part1, degp, b1r, W2)
  part2 = _sc_agg(y2, srcp, dstp)[0]
  q = _head(part2, degp, b2r, wlp, blp)
  outg = _sc_take(q, sampp)
  return outg[:N, :C]


# speedup vs baseline: 9.4989x; 1.0442x over previous
import jax
import jax.numpy as jnp
from jax import lax
from jax.experimental import pallas as pl
from jax.experimental.pallas import tpu as pltpu
from jax.experimental.pallas import tpu_sc as plsc

N = 10000
NP = 10240
D = 128
HW = D // 2
C = 40
E = 320000
EP = 327680
KC = 128
NC, NS = 2, 16
NCH = EP // (NS * KC)
RPT = NP // NS
BN = 1024


def _make_sc_agg(with_deg):
  mesh = plsc.VectorSubcoreMesh(core_axis_name="c", subcore_axis_name="s")
  out_type = [jax.ShapeDtypeStruct((NC, NP, HW), jnp.float32)]
  scratch = (
      [pltpu.VMEM((KC, HW), jnp.float32) for _ in range(4)]
      + [pltpu.VMEM((KC,), jnp.int32) for _ in range(4)]
      + [pltpu.VMEM((KC,), jnp.int32) for _ in range(4)]
      + [pltpu.VMEM_SHARED((NP, HW), jnp.float32)]
      + [pltpu.VMEM_SHARED((NP, HW), jnp.float32)]
      + [pltpu.SemaphoreType.DMA] * 16
  )
  if with_deg:
    out_type.append(jax.ShapeDtypeStruct((NC * NS * NP,), jnp.float32))
    scratch.append(pltpu.VMEM((NP,), jnp.float32))

  def body(y_hbm, src_hbm, dst_hbm, part_hbm, *rest):
    if with_deg:
      degp_hbm = rest[0]
      rest = rest[1:]
      hist = rest[30]
    else:
      hist = None
    rows = rest[0:4]
    srci = rest[4:8]
    dsti = rest[8:12]
    acc = rest[12]
    y_sp = rest[13]
    gsem = rest[14:18]
    ssem = rest[18:22]
    isems = rest[22:26]
    idems = rest[26:30]
    cid = lax.axis_index("c")
    sid = lax.axis_index("s")
    gw = cid * NS + sid
    base = sid * RPT
    ebase = sid * (EP // NS)

    pltpu.sync_copy(y_hbm.at[cid, pl.ds(base, RPT)],
                    y_sp.at[pl.ds(base, RPT)])

    @pl.loop(0, KC)
    def _(i):
      z = jnp.zeros((16,), jnp.float32)
      for j in range(HW // 16):
        rows[0][i, pl.ds(j * 16, 16)] = z

    for k in range(RPT // KC):
      pltpu.sync_copy(rows[0], acc.at[pl.ds(base + k * KC, KC)])

    if with_deg:
      @pl.loop(0, NP // 16)
      def _(i):
        hist[pl.ds(i * 16, 16)] = jnp.zeros((16,), jnp.float32)

    plsc.subcore_barrier()

    ones16 = jnp.ones((16,), jnp.float32)

    def count_deg(b):
      for j in range(KC // 16):
        idx = dsti[b][pl.ds(j * 16, 16)]
        plsc.addupdate_scatter(hist, [idx], ones16)

    def load_idx(b, c):
      off = ebase + c * KC
      pltpu.async_copy(src_hbm.at[pl.ds(off, KC)], srci[b], isems[b])
      pltpu.async_copy(dst_hbm.at[pl.ds(off, KC)], dsti[b], idems[b])

    def wait_idx(b):
      pltpu.make_async_copy(src_hbm.at[pl.ds(0, KC)], srci[b],
                            isems[b]).wait()
      pltpu.make_async_copy(dst_hbm.at[pl.ds(0, KC)], dsti[b],
                            idems[b]).wait()

    def issue_gather(b):
      pltpu.async_copy(y_sp.at[srci[b]], rows[b], gsem[b])

    def wait_gather(b):
      pltpu.make_async_copy(y_sp.at[srci[b]], rows[b], gsem[b]).wait()

    def issue_scatter(b):
      pltpu.async_copy(rows[b], acc.at[dsti[b]], ssem[b], add=True)

    def wait_scatter(b):
      pltpu.make_async_copy(rows[b], acc.at[dsti[b]], ssem[b]).wait()

    load_idx(0, 0)
    load_idx(1, 1)
    wait_idx(0)
    issue_gather(0)

    @pl.loop(0, NCH // 4)
    def _(p):
      for i in range(4):
        s = 4 * p + i
        b0 = i
        b1 = (i + 1) % 4
        b2 = (i + 2) % 4

        @pl.when(s >= 2)
        def _():
          wait_scatter(b2)

        @pl.when(s + 2 < NCH)
        def _():
          load_idx(b2, s + 2)

        @pl.when(s + 1 < NCH)
        def _():
          wait_idx(b1)
          issue_gather(b1)

        wait_gather(b0)
        issue_scatter(b0)
        if with_deg:
          count_deg(b0)

    wait_scatter((NCH - 2) % 4)
    wait_scatter((NCH - 1) % 4)

    plsc.subcore_barrier()

    for k in range(RPT // KC):
      r = base + k * KC
      pltpu.sync_copy(acc.at[pl.ds(r, KC)], part_hbm.at[cid, pl.ds(r, KC)])
    if with_deg:
      pltpu.sync_copy(hist, degp_hbm.at[pl.ds(gw * NP, NP)])

  return pl.kernel(
      body, out_type=tuple(out_type), mesh=mesh,
      scratch_types=tuple(scratch),
      compiler_params=pltpu.CompilerParams(
          needs_layout_passes=False, use_tc_tiling_on_sc=False))


_sc_agg_deg = _make_sc_agg(True)
_sc_agg = _make_sc_agg(False)


GCH = 8
GK = 40


def _sc_take_body(q_hbm, samp_hbm, out_hbm, sampv, rows_a, rows_b, q_sp,
                  sem_a, sem_b):
  cid = lax.axis_index("c")
  sid = lax.axis_index("s")
  gw = cid * NS + sid
  base = sid * RPT
  pltpu.sync_copy(q_hbm.at[pl.ds(base, RPT)], q_sp.at[pl.ds(base, RPT)])
  pltpu.sync_copy(samp_hbm.at[pl.ds(gw * GCH, GCH)], sampv)
  plsc.subcore_barrier()
  bufs = [(rows_a, sem_a), (rows_b, sem_b)]
  pltpu.async_copy(q_sp.at[sampv.at[0]], rows_a, sem_a)
  for c in range(GCH):
    buf, sem = bufs[c % 2]
    pltpu.make_async_copy(q_sp.at[sampv.at[c]], buf, sem).wait()
    if c + 1 < GCH:
      nbuf, nsem = bufs[(c + 1) % 2]
      pltpu.async_copy(q_sp.at[sampv.at[c + 1]], nbuf, nsem)
    pltpu.sync_copy(buf, out_hbm.at[pl.ds((gw * GCH + c) * GK, GK)])


_sc_take = pl.kernel(
    _sc_take_body,
    out_type=jax.ShapeDtypeStruct((NP, D), jnp.float32),
    mesh=plsc.VectorSubcoreMesh(core_axis_name="c", subcore_axis_name="s"),
    scratch_types=(
        pltpu.VMEM((GCH, GK), jnp.int32),
        pltpu.VMEM((GK, D), jnp.float32),
        pltpu.VMEM((GK, D), jnp.float32),
        pltpu.VMEM_SHARED((NP, D), jnp.float32),
        pltpu.SemaphoreType.DMA,
        pltpu.SemaphoreType.DMA,
    ),
    compiler_params=pltpu.CompilerParams(
        needs_layout_passes=False, use_tc_tiling_on_sc=False))


def _split_cols(y):
  return jnp.stack([y[:, :HW], y[:, HW:]])


def _mm_body(x_ref, w_ref, o_ref):
  y = jnp.dot(x_ref[...], w_ref[...], preferred_element_type=jnp.float32)
  o_ref[...] = _split_cols(y)


def _mm(x, w):
  return pl.pallas_call(
      _mm_body,
      grid=(NP // BN,),
      in_specs=[pl.BlockSpec((BN, D), lambda i: (i, 0)),
                pl.BlockSpec((D, D), lambda i: (0, 0))],
      out_specs=pl.BlockSpec((NC, BN, HW), lambda i: (0, i, 0)),
      out_shape=jax.ShapeDtypeStruct((NC, NP, HW), jnp.float32),
  )(x, w)


def _agg_to_h(p_ref, dg_ref, b_ref):
  agg = jnp.concatenate([p_ref[0], p_ref[1]], axis=1)
  deg = jnp.maximum(0.5 * jnp.sum(dg_ref[...], axis=0), 1.0)[:, None]
  return jnp.maximum(agg / deg + b_ref[...], 0.0)


def _layer_body(p_ref, dg_ref, b_ref, w_ref, o_ref):
  h = _agg_to_h(p_ref, dg_ref, b_ref)
  y = jnp.dot(h, w_ref[...], preferred_element_type=jnp.float32)
  o_ref[...] = _split_cols(y)


def _head_body(p_ref, dg_ref, b_ref, w_ref, bl_ref, o_ref):
  h = _agg_to_h(p_ref, dg_ref, b_ref)
  nrm = jnp.sqrt(jnp.sum(h * h, axis=1, keepdims=True))
  g = h / jnp.maximum(nrm, 1e-12)
  o_ref[...] = jnp.dot(g, w_ref[...],
                       preferred_element_type=jnp.float32) + bl_ref[...]


def _layer(part, degp, b, w):
  return pl.pallas_call(
      _layer_body,
      grid=(NP // BN,),
      in_specs=[pl.BlockSpec((NC, BN, HW), lambda i: (0, i, 0)),
                pl.BlockSpec((NC * NS, BN), lambda i: (0, i)),
                pl.BlockSpec((1, D), lambda i: (0, 0)),
                pl.BlockSpec((D, D), lambda i: (0, 0))],
      out_specs=pl.BlockSpec((NC, BN, HW), lambda i: (0, i, 0)),
      out_shape=jax.ShapeDtypeStruct((NC, NP, HW), jnp.float32),
  )(part, degp, b, w)


def _head(part, degp, b, w, bl):
  return pl.pallas_call(
      _head_body,
      grid=(NP // BN,),
      in_specs=[pl.BlockSpec((NC, BN, HW), lambda i: (0, i, 0)),
                pl.BlockSpec((NC * NS, BN), lambda i: (0, i)),
                pl.BlockSpec((1, D), lambda i: (0, 0)),
                pl.BlockSpec((D, D), lambda i: (0, 0)),
                pl.BlockSpec((1, D), lambda i: (0, 0))],
      out_specs=pl.BlockSpec((BN, D), lambda i: (i, 0)),
      out_shape=jax.ShapeDtypeStruct((NP, D), jnp.float32),
  )(part, degp, b, w, bl)


def kernel(feat, adjs, sampled_nodes, nodes_per_layer, iterations,
           W1, b1, W2, b2, Wlin, blin):
  f32 = jnp.float32

  src = adjs[0]
  dst = adjs[1]
  srcp = jnp.concatenate([src, jnp.zeros((EP - E,), jnp.int32)])
  dstp = jnp.concatenate([dst, jnp.full((EP - E,), NP - 1, jnp.int32)])
  sampp = jnp.concatenate(
      [sampled_nodes, jnp.zeros((NP - N,), jnp.int32)]).reshape(32 * GCH, GK)
  b1r = b1.reshape(1, D)
  b2r = b2.reshape(1, D)
  wlp = jnp.zeros((D, D), f32).at[:, :C].set(Wlin)
  blp = jnp.zeros((1, D), f32).at[0, :C].set(blin)

  y1 = _mm(feat, W1)
  part1, degp = _sc_agg_deg(y1, srcp, dstp)
  degp = degp.reshape(NC * NS, NP)
  y2 = _layer(part1, degp, b1r, W2)
  part2 = _sc_agg(y2, srcp, dstp)[0]
  q = _head(part2, degp, b2r, wlp, blp)
  outg = _sc_take(q, sampp)
  return outg[:N, :C]


# speedup vs baseline: 9.9395x; 1.0464x over previous
import jax
import jax.numpy as jnp
from jax import lax
from jax.experimental import pallas as pl
from jax.experimental.pallas import tpu as pltpu
from jax.experimental.pallas import tpu_sc as plsc

N = 10000
NP = 10240
D = 128
HW = D // 2
C = 40
E = 320000
EPT = E // 16
KC = 128
TKC = 32
NC, NS = 2, 16
NCH = 156
RPT = NP // NS
BN = 1024


def _make_sc_agg(with_deg):
  mesh = plsc.VectorSubcoreMesh(core_axis_name="c", subcore_axis_name="s")
  out_type = [jax.ShapeDtypeStruct((NC, NP, HW), jnp.float32)]
  scratch = (
      [pltpu.VMEM((KC, HW), jnp.float32) for _ in range(4)]
      + [pltpu.VMEM((KC,), jnp.int32) for _ in range(4)]
      + [pltpu.VMEM((KC,), jnp.int32) for _ in range(4)]
      + [pltpu.VMEM_SHARED((NP, HW), jnp.float32)]
      + [pltpu.VMEM_SHARED((NP, HW), jnp.float32)]
      + [pltpu.SemaphoreType.DMA] * 16
      + [pltpu.VMEM((TKC,), jnp.int32)] * 2
  )
  if with_deg:
    out_type.append(jax.ShapeDtypeStruct((NC * NS * NP,), jnp.float32))
    scratch.append(pltpu.VMEM((NP,), jnp.float32))

  def body(y_hbm, src_hbm, dst_hbm, part_hbm, *rest):
    if with_deg:
      degp_hbm = rest[0]
      rest = rest[1:]
      hist = rest[32]
    else:
      hist = None
    srct, dstt = rest[30], rest[31]
    rows = rest[0:4]
    srci = rest[4:8]
    dsti = rest[8:12]
    acc = rest[12]
    y_sp = rest[13]
    gsem = rest[14:18]
    ssem = rest[18:22]
    isems = rest[22:26]
    idems = rest[26:30]
    cid = lax.axis_index("c")
    sid = lax.axis_index("s")
    gw = cid * NS + sid
    base = sid * RPT
    ebase = sid * EPT

    pltpu.sync_copy(y_hbm.at[cid, pl.ds(base, RPT)],
                    y_sp.at[pl.ds(base, RPT)])

    @pl.loop(0, KC)
    def _(i):
      z = jnp.zeros((16,), jnp.float32)
      for j in range(HW // 16):
        rows[0][i, pl.ds(j * 16, 16)] = z

    for k in range(RPT // KC):
      pltpu.sync_copy(rows[0], acc.at[pl.ds(base + k * KC, KC)])

    if with_deg:
      @pl.loop(0, NP // 16)
      def _(i):
        hist[pl.ds(i * 16, 16)] = jnp.zeros((16,), jnp.float32)

    plsc.subcore_barrier()

    ones16 = jnp.ones((16,), jnp.float32)

    def count_deg(b):
      for j in range(KC // 16):
        idx = dsti[b][pl.ds(j * 16, 16)]
        plsc.addupdate_scatter(hist, [idx], ones16)

    def load_idx(b, c):
      off = ebase + c * KC
      pltpu.async_copy(src_hbm.at[pl.ds(off, KC)], srci[b], isems[b])
      pltpu.async_copy(dst_hbm.at[pl.ds(off, KC)], dsti[b], idems[b])

    def wait_idx(b):
      pltpu.make_async_copy(src_hbm.at[pl.ds(0, KC)], srci[b],
                            isems[b]).wait()
      pltpu.make_async_copy(dst_hbm.at[pl.ds(0, KC)], dsti[b],
                            idems[b]).wait()

    def issue_gather(b):
      pltpu.async_copy(y_sp.at[srci[b]], rows[b], gsem[b])

    def wait_gather(b):
      pltpu.make_async_copy(y_sp.at[srci[b]], rows[b], gsem[b]).wait()

    def issue_scatter(b):
      pltpu.async_copy(rows[b], acc.at[dsti[b]], ssem[b], add=True)

    def wait_scatter(b):
      pltpu.make_async_copy(rows[b], acc.at[dsti[b]], ssem[b]).wait()

    load_idx(0, 0)
    load_idx(1, 1)
    wait_idx(0)
    issue_gather(0)

    @pl.loop(0, NCH // 4)
    def _(p):
      for i in range(4):
        s = 4 * p + i
        b0 = i
        b1 = (i + 1) % 4
        b2 = (i + 2) % 4

        @pl.when(s >= 2)
        def _():
          wait_scatter(b2)

        @pl.when(s + 2 < NCH)
        def _():
          load_idx(b2, s + 2)

        @pl.when(s + 1 < NCH)
        def _():
          wait_idx(b1)
          issue_gather(b1)

        wait_gather(b0)
        issue_scatter(b0)
        if with_deg:
          count_deg(b0)

    wait_scatter((NCH - 2) % 4)
    wait_scatter((NCH - 1) % 4)

    toff = ebase + NCH * KC
    pltpu.async_copy(src_hbm.at[pl.ds(toff, TKC)], srct, isems[0])
    pltpu.async_copy(dst_hbm.at[pl.ds(toff, TKC)], dstt, idems[0])
    pltpu.make_async_copy(src_hbm.at[pl.ds(0, TKC)], srct, isems[0]).wait()
    pltpu.make_async_copy(dst_hbm.at[pl.ds(0, TKC)], dstt, idems[0]).wait()
    trows = rows[0].at[pl.ds(0, TKC)]
    pltpu.async_copy(y_sp.at[srct], trows, gsem[0])
    pltpu.make_async_copy(y_sp.at[srct], trows, gsem[0]).wait()
    pltpu.sync_copy(trows, acc.at[dstt], add=True)
    if with_deg:
      for j in range(TKC // 16):
        idx = dstt[pl.ds(j * 16, 16)]
        plsc.addupdate_scatter(hist, [idx], ones16)

    plsc.subcore_barrier()

    for k in range(RPT // KC):
      r = base + k * KC
      pltpu.sync_copy(acc.at[pl.ds(r, KC)], part_hbm.at[cid, pl.ds(r, KC)])
    if with_deg:
      pltpu.sync_copy(hist, degp_hbm.at[pl.ds(gw * NP, NP)])

  return pl.kernel(
      body, out_type=tuple(out_type), mesh=mesh,
      scratch_types=tuple(scratch),
      compiler_params=pltpu.CompilerParams(
          needs_layout_passes=False, use_tc_tiling_on_sc=False))


_sc_agg_deg = _make_sc_agg(True)
_sc_agg = _make_sc_agg(False)


GCH = 8
GK = 40


def _sc_take_body(q_hbm, samp_hbm, out_hbm, sampv, rows_a, rows_b, q_sp,
                  sem_a, sem_b):
  cid = lax.axis_index("c")
  sid = lax.axis_index("s")
  gw = cid * NS + sid
  base = sid * RPT
  pltpu.sync_copy(q_hbm.at[pl.ds(base, RPT)], q_sp.at[pl.ds(base, RPT)])
  pltpu.sync_copy(samp_hbm.at[pl.ds(gw * GCH, GCH)], sampv)
  plsc.subcore_barrier()
  bufs = [(rows_a, sem_a), (rows_b, sem_b)]
  pltpu.async_copy(q_sp.at[sampv.at[0]], rows_a, sem_a)
  for c in range(GCH):
    buf, sem = bufs[c % 2]
    pltpu.make_async_copy(q_sp.at[sampv.at[c]], buf, sem).wait()
    if c + 1 < GCH:
      nbuf, nsem = bufs[(c + 1) % 2]
      pltpu.async_copy(q_sp.at[sampv.at[c + 1]], nbuf, nsem)
    pltpu.sync_copy(buf, out_hbm.at[pl.ds((gw * GCH + c) * GK, GK)])


_sc_take = pl.kernel(
    _sc_take_body,
    out_type=jax.ShapeDtypeStruct((NP, D), jnp.float32),
    mesh=plsc.VectorSubcoreMesh(core_axis_name="c", subcore_axis_name="s"),
    scratch_types=(
        pltpu.VMEM((GCH, GK), jnp.int32),
        pltpu.VMEM((GK, D), jnp.float32),
        pltpu.VMEM((GK, D), jnp.float32),
        pltpu.VMEM_SHARED((NP, D), jnp.float32),
        pltpu.SemaphoreType.DMA,
        pltpu.SemaphoreType.DMA,
    ),
    compiler_params=pltpu.CompilerParams(
        needs_layout_passes=False, use_tc_tiling_on_sc=False))


def _split_cols(y):
  return jnp.stack([y[:, :HW], y[:, HW:]])


def _mm_body(x_ref, w_ref, o_ref):
  y = jnp.dot(x_ref[...], w_ref[...], preferred_element_type=jnp.float32)
  o_ref[...] = _split_cols(y)


def _mm(x, w):
  return pl.pallas_call(
      _mm_body,
      grid=(NP // BN,),
      in_specs=[pl.BlockSpec((BN, D), lambda i: (i, 0)),
                pl.BlockSpec((D, D), lambda i: (0, 0))],
      out_specs=pl.BlockSpec((NC, BN, HW), lambda i: (0, i, 0)),
      out_shape=jax.ShapeDtypeStruct((NC, NP, HW), jnp.float32),
  )(x, w)


def _agg_to_h(p_ref, dg_ref, b_ref):
  agg = jnp.concatenate([p_ref[0], p_ref[1]], axis=1)
  deg = jnp.maximum(0.5 * jnp.sum(dg_ref[...], axis=0), 1.0)[:, None]
  return jnp.maximum(agg / deg + b_ref[...], 0.0)


def _layer_body(p_ref, dg_ref, b_ref, w_ref, o_ref):
  h = _agg_to_h(p_ref, dg_ref, b_ref)
  y = jnp.dot(h, w_ref[...], preferred_element_type=jnp.float32)
  o_ref[...] = _split_cols(y)


def _head_body(p_ref, dg_ref, b_ref, w_ref, bl_ref, o_ref):
  h = _agg_to_h(p_ref, dg_ref, b_ref)
  nrm = jnp.sqrt(jnp.sum(h * h, axis=1, keepdims=True))
  g = h / jnp.maximum(nrm, 1e-12)
  o_ref[...] = jnp.dot(g, w_ref[...],
                       preferred_element_type=jnp.float32) + bl_ref[...]


def _layer(part, degp, b, w):
  return pl.pallas_call(
      _layer_body,
      grid=(NP // BN,),
      in_specs=[pl.BlockSpec((NC, BN, HW), lambda i: (0, i, 0)),
                pl.BlockSpec((NC * NS, BN), lambda i: (0, i)),
                pl.BlockSpec((1, D), lambda i: (0, 0)),
                pl.BlockSpec((D, D), lambda i: (0, 0))],
      out_specs=pl.BlockSpec((NC, BN, HW), lambda i: (0, i, 0)),
      out_shape=jax.ShapeDtypeStruct((NC, NP, HW), jnp.float32),
  )(part, degp, b, w)


def _head(part, degp, b, w, bl):
  return pl.pallas_call(
      _head_body,
      grid=(NP // BN,),
      in_specs=[pl.BlockSpec((NC, BN, HW), lambda i: (0, i, 0)),
                pl.BlockSpec((NC * NS, BN), lambda i: (0, i)),
                pl.BlockSpec((1, D), lambda i: (0, 0)),
                pl.BlockSpec((D, D), lambda i: (0, 0)),
                pl.BlockSpec((1, D), lambda i: (0, 0))],
      out_specs=pl.BlockSpec((BN, D), lambda i: (i, 0)),
      out_shape=jax.ShapeDtypeStruct((NP, D), jnp.float32),
  )(part, degp, b, w, bl)


def kernel(feat, adjs, sampled_nodes, nodes_per_layer, iterations,
           W1, b1, W2, b2, Wlin, blin):
  f32 = jnp.float32

  srcp = adjs[0]
  dstp = adjs[1]
  sampp = jnp.concatenate(
      [sampled_nodes, jnp.zeros((NP - N,), jnp.int32)]).reshape(32 * GCH, GK)
  b1r = b1.reshape(1, D)
  b2r = b2.reshape(1, D)
  wlp = jnp.zeros((D, D), f32).at[:, :C].set(Wlin)
  blp = jnp.zeros((1, D), f32).at[0, :C].set(blin)

  y1 = _mm(feat, W1)
  part1, degp = _sc_agg_deg(y1, srcp, dstp)
  degp = degp.reshape(NC * NS, NP)
  y2 = _layer(part1, degp, b1r, W2)
  part2 = _sc_agg(y2, srcp, dstp)[0]
  q = _head(part2, degp, b2r, wlp, blp)
  outg = _sc_take(q, sampp)
  return outg[:N, :C]


# speedup vs baseline: 10.1804x; 1.0242x over previous
import jax
import jax.numpy as jnp
from jax import lax
from jax.experimental import pallas as pl
from jax.experimental.pallas import tpu as pltpu
from jax.experimental.pallas import tpu_sc as plsc

N = 10000
NP = 10240
D = 128
HW = D // 2
C = 40
E = 320000
EPT = E // 16
KC = 128
TKC = 32
NC, NS = 2, 16
NCH = 156
RPT = NP // NS
BN = 10240


def _make_sc_agg(with_deg):
  mesh = plsc.VectorSubcoreMesh(core_axis_name="c", subcore_axis_name="s")
  out_type = [jax.ShapeDtypeStruct((NC, NP, HW), jnp.float32)]
  scratch = (
      [pltpu.VMEM((KC, HW), jnp.float32) for _ in range(4)]
      + [pltpu.VMEM((KC,), jnp.int32) for _ in range(4)]
      + [pltpu.VMEM((KC,), jnp.int32) for _ in range(4)]
      + [pltpu.VMEM_SHARED((NP, HW), jnp.float32)]
      + [pltpu.VMEM_SHARED((NP, HW), jnp.float32)]
      + [pltpu.SemaphoreType.DMA] * 16
      + [pltpu.VMEM((TKC,), jnp.int32)] * 2
  )
  if with_deg:
    out_type.append(jax.ShapeDtypeStruct((NC * NS * NP,), jnp.float32))
    scratch.append(pltpu.VMEM((NP,), jnp.float32))

  def body(y_hbm, src_hbm, dst_hbm, part_hbm, *rest):
    if with_deg:
      degp_hbm = rest[0]
      rest = rest[1:]
      hist = rest[32]
    else:
      hist = None
    srct, dstt = rest[30], rest[31]
    rows = rest[0:4]
    srci = rest[4:8]
    dsti = rest[8:12]
    acc = rest[12]
    y_sp = rest[13]
    gsem = rest[14:18]
    ssem = rest[18:22]
    isems = rest[22:26]
    idems = rest[26:30]
    cid = lax.axis_index("c")
    sid = lax.axis_index("s")
    gw = cid * NS + sid
    base = sid * RPT
    ebase = sid * EPT

    pltpu.sync_copy(y_hbm.at[cid, pl.ds(base, RPT)],
                    y_sp.at[pl.ds(base, RPT)])

    @pl.loop(0, KC)
    def _(i):
      z = jnp.zeros((16,), jnp.float32)
      for j in range(HW // 16):
        rows[0][i, pl.ds(j * 16, 16)] = z

    for k in range(RPT // KC):
      pltpu.sync_copy(rows[0], acc.at[pl.ds(base + k * KC, KC)])

    if with_deg:
      @pl.loop(0, NP // 16)
      def _(i):
        hist[pl.ds(i * 16, 16)] = jnp.zeros((16,), jnp.float32)

    plsc.subcore_barrier()

    ones16 = jnp.ones((16,), jnp.float32)

    def count_deg(b):
      for j in range(KC // 16):
        idx = dsti[b][pl.ds(j * 16, 16)]
        plsc.addupdate_scatter(hist, [idx], ones16)

    def load_idx(b, c):
      off = ebase + c * KC
      pltpu.async_copy(src_hbm.at[pl.ds(off, KC)], srci[b], isems[b])
      pltpu.async_copy(dst_hbm.at[pl.ds(off, KC)], dsti[b], idems[b])

    def wait_idx(b):
      pltpu.make_async_copy(src_hbm.at[pl.ds(0, KC)], srci[b],
                            isems[b]).wait()
      pltpu.make_async_copy(dst_hbm.at[pl.ds(0, KC)], dsti[b],
                            idems[b]).wait()

    def issue_gather(b):
      pltpu.async_copy(y_sp.at[srci[b]], rows[b], gsem[b])

    def wait_gather(b):
      pltpu.make_async_copy(y_sp.at[srci[b]], rows[b], gsem[b]).wait()

    def issue_scatter(b):
      pltpu.async_copy(rows[b], acc.at[dsti[b]], ssem[b], add=True)

    def wait_scatter(b):
      pltpu.make_async_copy(rows[b], acc.at[dsti[b]], ssem[b]).wait()

    load_idx(0, 0)
    load_idx(1, 1)
    wait_idx(0)
    issue_gather(0)

    @pl.loop(0, NCH // 4)
    def _(p):
      for i in range(4):
        s = 4 * p + i
        b0 = i
        b1 = (i + 1) % 4
        b2 = (i + 2) % 4

        @pl.when(s >= 2)
        def _():
          wait_scatter(b2)

        @pl.when(s + 2 < NCH)
        def _():
          load_idx(b2, s + 2)

        @pl.when(s + 1 < NCH)
        def _():
          wait_idx(b1)
          issue_gather(b1)

        wait_gather(b0)
        issue_scatter(b0)
        if with_deg:
          count_deg(b0)

    wait_scatter((NCH - 2) % 4)
    wait_scatter((NCH - 1) % 4)

    toff = ebase + NCH * KC
    pltpu.async_copy(src_hbm.at[pl.ds(toff, TKC)], srct, isems[0])
    pltpu.async_copy(dst_hbm.at[pl.ds(toff, TKC)], dstt, idems[0])
    pltpu.make_async_copy(src_hbm.at[pl.ds(0, TKC)], srct, isems[0]).wait()
    pltpu.make_async_copy(dst_hbm.at[pl.ds(0, TKC)], dstt, idems[0]).wait()
    trows = rows[0].at[pl.ds(0, TKC)]
    pltpu.async_copy(y_sp.at[srct], trows, gsem[0])
    pltpu.make_async_copy(y_sp.at[srct], trows, gsem[0]).wait()
    pltpu.sync_copy(trows, acc.at[dstt], add=True)
    if with_deg:
      for j in range(TKC // 16):
        idx = dstt[pl.ds(j * 16, 16)]
        plsc.addupdate_scatter(hist, [idx], ones16)

    plsc.subcore_barrier()

    for k in range(RPT // KC):
      r = base + k * KC
      pltpu.sync_copy(acc.at[pl.ds(r, KC)], part_hbm.at[cid, pl.ds(r, KC)])
    if with_deg:
      pltpu.sync_copy(hist, degp_hbm.at[pl.ds(gw * NP, NP)])

  return pl.kernel(
      body, out_type=tuple(out_type), mesh=mesh,
      scratch_types=tuple(scratch),
      compiler_params=pltpu.CompilerParams(
          needs_layout_passes=False, use_tc_tiling_on_sc=False))


_sc_agg_deg = _make_sc_agg(True)
_sc_agg = _make_sc_agg(False)


GCH = 8
GK = 40


def _sc_take_body(q_hbm, samp_hbm, out_hbm, sampv, rows_a, rows_b, q_sp,
                  sem_a, sem_b):
  cid = lax.axis_index("c")
  sid = lax.axis_index("s")
  gw = cid * NS + sid
  base = sid * RPT
  pltpu.sync_copy(q_hbm.at[pl.ds(base, RPT)], q_sp.at[pl.ds(base, RPT)])
  pltpu.sync_copy(samp_hbm.at[pl.ds(gw * GCH, GCH)], sampv)
  plsc.subcore_barrier()
  bufs = [(rows_a, sem_a), (rows_b, sem_b)]
  pltpu.async_copy(q_sp.at[sampv.at[0]], rows_a, sem_a)
  for c in range(GCH):
    buf, sem = bufs[c % 2]
    pltpu.make_async_copy(q_sp.at[sampv.at[c]], buf, sem).wait()
    if c + 1 < GCH:
      nbuf, nsem = bufs[(c + 1) % 2]
      pltpu.async_copy(q_sp.at[sampv.at[c + 1]], nbuf, nsem)
    pltpu.sync_copy(buf, out_hbm.at[pl.ds((gw * GCH + c) * GK, GK)])


_sc_take = pl.kernel(
    _sc_take_body,
    out_type=jax.ShapeDtypeStruct((NP, D), jnp.float32),
    mesh=plsc.VectorSubcoreMesh(core_axis_name="c", subcore_axis_name="s"),
    scratch_types=(
        pltpu.VMEM((GCH, GK), jnp.int32),
        pltpu.VMEM((GK, D), jnp.float32),
        pltpu.VMEM((GK, D), jnp.float32),
        pltpu.VMEM_SHARED((NP, D), jnp.float32),
        pltpu.SemaphoreType.DMA,
        pltpu.SemaphoreType.DMA,
    ),
    compiler_params=pltpu.CompilerParams(
        needs_layout_passes=False, use_tc_tiling_on_sc=False))


def _split_cols(y):
  return jnp.stack([y[:, :HW], y[:, HW:]])


def _mm_body(x_ref, w_ref, o_ref):
  y = jnp.dot(x_ref[...], w_ref[...], preferred_element_type=jnp.float32)
  o_ref[...] = _split_cols(y)


def _mm(x, w):
  return pl.pallas_call(
      _mm_body,
      grid=(NP // BN,),
      in_specs=[pl.BlockSpec((BN, D), lambda i: (i, 0)),
                pl.BlockSpec((D, D), lambda i: (0, 0))],
      out_specs=pl.BlockSpec((NC, BN, HW), lambda i: (0, i, 0)),
      out_shape=jax.ShapeDtypeStruct((NC, NP, HW), jnp.float32),
  )(x, w)


def _agg_to_h(p_ref, dg_ref, b_ref):
  agg = jnp.concatenate([p_ref[0], p_ref[1]], axis=1)
  deg = jnp.maximum(0.5 * jnp.sum(dg_ref[...], axis=0), 1.0)[:, None]
  return jnp.maximum(agg / deg + b_ref[...], 0.0)


def _layer_body(p_ref, dg_ref, b_ref, w_ref, o_ref):
  h = _agg_to_h(p_ref, dg_ref, b_ref)
  y = jnp.dot(h, w_ref[...], preferred_element_type=jnp.float32)
  o_ref[...] = _split_cols(y)


def _head_body(p_ref, dg_ref, b_ref, w_ref, bl_ref, o_ref):
  h = _agg_to_h(p_ref, dg_ref, b_ref)
  nrm = jnp.sqrt(jnp.sum(h * h, axis=1, keepdims=True))
  g = h / jnp.maximum(nrm, 1e-12)
  o_ref[...] = jnp.dot(g, w_ref[...],
                       preferred_element_type=jnp.float32) + bl_ref[...]


def _layer(part, degp, b, w):
  return pl.pallas_call(
      _layer_body,
      grid=(NP // BN,),
      in_specs=[pl.BlockSpec((NC, BN, HW), lambda i: (0, i, 0)),
                pl.BlockSpec((NC * NS, BN), lambda i: (0, i)),
                pl.BlockSpec((1, D), lambda i: (0, 0)),
                pl.BlockSpec((D, D), lambda i: (0, 0))],
      out_specs=pl.BlockSpec((NC, BN, HW), lambda i: (0, i, 0)),
      out_shape=jax.ShapeDtypeStruct((NC, NP, HW), jnp.float32),
  )(part, degp, b, w)


def _head(part, degp, b, w, bl):
  return pl.pallas_call(
      _head_body,
      grid=(NP // BN,),
      in_specs=[pl.BlockSpec((NC, BN, HW), lambda i: (0, i, 0)),
                pl.BlockSpec((NC * NS, BN), lambda i: (0, i)),
                pl.BlockSpec((1, D), lambda i: (0, 0)),
                pl.BlockSpec((D, D), lambda i: (0, 0)),
                pl.BlockSpec((1, D), lambda i: (0, 0))],
      out_specs=pl.BlockSpec((BN, D), lambda i: (i, 0)),
      out_shape=jax.ShapeDtypeStruct((NP, D), jnp.float32),
  )(part, degp, b, w, bl)


def kernel(feat, adjs, sampled_nodes, nodes_per_layer, iterations,
           W1, b1, W2, b2, Wlin, blin):
  f32 = jnp.float32

  srcp = adjs[0]
  dstp = adjs[1]
  sampp = jnp.concatenate(
      [sampled_nodes, jnp.zeros((NP - N,), jnp.int32)]).reshape(32 * GCH, GK)
  b1r = b1.reshape(1, D)
  b2r = b2.reshape(1, D)
  wlp = jnp.zeros((D, D), f32).at[:, :C].set(Wlin)
  blp = jnp.zeros((1, D), f32).at[0, :C].set(blin)

  y1 = _mm(feat, W1)
  part1, degp = _sc_agg_deg(y1, srcp, dstp)
  degp = degp.reshape(NC * NS, NP)
  y2 = _layer(part1, degp, b1r, W2)
  part2 = _sc_agg(y2, srcp, dstp)[0]
  q = _head(part2, degp, b2r, wlp, blp)
  outg = _sc_take(q, sampp)
  return outg[:N, :C]
